# Initial kernel scaffold; baseline (speedup 1.0000x reference)
import jax
import jax.numpy as jnp
from jax.experimental import pallas as pl


def kernel(features, edge_index, W1, W2):
    raise NotImplementedError("write your pallas kernel here")


# speedup vs baseline: 23.5525x; 23.5525x over previous
import functools

import jax
import jax.numpy as jnp
from jax import lax
from jax.experimental import pallas as pl
from jax.experimental.pallas import tpu as pltpu
from jax.experimental.pallas import tpu_sc as plsc

NC = 2
NS = 16
NW = NC * NS
CHUNK = 80
NODE_BLK = 1024


def _edge_pass(n_pad, h, e):
  epw = e // NW
  nchunk = epw // CHUNK
  rpt = n_pad // NS
  mesh = plsc.VectorSubcoreMesh(core_axis_name="c", subcore_axis_name="s")

  @functools.partial(
      pl.kernel,
      mesh=mesh,
      out_type=jax.ShapeDtypeStruct((NC, n_pad, h), jnp.float32),
      scratch_types=[
          pltpu.VMEM((nchunk, CHUNK), jnp.int32),
          pltpu.VMEM((nchunk, CHUNK), jnp.int32),
          pltpu.VMEM((CHUNK, h), jnp.float32),
          pltpu.VMEM_SHARED((n_pad, h), jnp.float32),
          pltpu.SemaphoreType.DMA,
      ],
      compiler_params=pltpu.CompilerParams(use_tc_tiling_on_sc=False),
  )
  def body(table_hbm, src_hbm, dst_hbm, zeros_hbm, out_hbm,
           sidx, didx, rows, acc, sem):
    cid = lax.axis_index("c")
    sid = lax.axis_index("s")
    wid = sid * NC + cid
    pltpu.sync_copy(src_hbm.at[wid], sidx)
    pltpu.sync_copy(dst_hbm.at[wid], didx)
    pltpu.sync_copy(zeros_hbm.at[pl.ds(sid * rpt, rpt)],
                    acc.at[pl.ds(sid * rpt, rpt)])
    plsc.subcore_barrier()

    def step(c, carry):
      pltpu.async_copy(table_hbm.at[sidx.at[c]], rows, sem).wait()
      pltpu.sync_copy(rows, acc.at[didx.at[c]], add=True)
      return carry

    lax.fori_loop(0, nchunk, step, 0)
    plsc.subcore_barrier()
    pltpu.sync_copy(acc.at[pl.ds(sid * rpt, rpt)],
                    out_hbm.at[cid, pl.ds(sid * rpt, rpt)])

  return body


def _deg_pass(n_pad, e):
  epw = e // NW
  nchunk = epw // CHUNK
  rpt = n_pad // NS
  mesh = plsc.VectorSubcoreMesh(core_axis_name="c", subcore_axis_name="s")

  @functools.partial(
      pl.kernel,
      mesh=mesh,
      out_type=jax.ShapeDtypeStruct((NC, n_pad), jnp.float32),
      scratch_types=[
          pltpu.VMEM((nchunk, CHUNK), jnp.int32),
          pltpu.VMEM((CHUNK,), jnp.float32),
          pltpu.VMEM_SHARED((n_pad,), jnp.float32),
      ],
      compiler_params=pltpu.CompilerParams(use_tc_tiling_on_sc=False),
  )
  def body(dst_hbm, zeros_hbm, out_hbm, didx, ones, acc):
    cid = lax.axis_index("c")
    sid = lax.axis_index("s")
    wid = sid * NC + cid
    for k in range(CHUNK // 16):
      ones[pl.ds(k * 16, 16)] = jnp.full((16,), 1.0, jnp.float32)
    pltpu.sync_copy(dst_hbm.at[wid], didx)
    pltpu.sync_copy(zeros_hbm.at[pl.ds(sid * rpt, rpt)],
                    acc.at[pl.ds(sid * rpt, rpt)])
    plsc.subcore_barrier()

    def step(c, carry):
      pltpu.sync_copy(ones, acc.at[didx.at[c]], add=True)
      return carry

    lax.fori_loop(0, nchunk, step, 0)
    plsc.subcore_barrier()
    pltpu.sync_copy(acc.at[pl.ds(sid * rpt, rpt)],
                    out_hbm.at[cid, pl.ds(sid * rpt, rpt)])

  return body


def _tc_scale_matmul(n_pad, d, h):
  nblk = n_pad // NODE_BLK

  def body(x_ref, w_ref, dp_ref, hs_ref, dinv_ref):
    deg = dp_ref[0] + dp_ref[1] + 1.0
    dinv = lax.rsqrt(jnp.maximum(deg, 1.0))
    hmat = jnp.dot(x_ref[...], w_ref[...], preferred_element_type=jnp.float32)
    hs_ref[...] = hmat * dinv
    dinv_ref[...] = dinv

  return pl.pallas_call(
      body,
      grid=(nblk,),
      in_specs=[
          pl.BlockSpec((NODE_BLK, d), lambda i: (i, 0)),
          pl.BlockSpec((d, h), lambda i: (0, 0)),
          pl.BlockSpec((NC, NODE_BLK, 1), lambda i: (0, i, 0)),
      ],
      out_specs=[
          pl.BlockSpec((NODE_BLK, h), lambda i: (i, 0)),
          pl.BlockSpec((NODE_BLK, 1), lambda i: (i, 0)),
      ],
      out_shape=[
          jax.ShapeDtypeStruct((n_pad, h), jnp.float32),
          jax.ShapeDtypeStruct((n_pad, 1), jnp.float32),
      ],
  )


def _tc_mid(n_pad, h, out_w):
  nblk = n_pad // NODE_BLK

  def body(q_ref, hs_ref, dinv_ref, w_ref, out_ref):
    acc = q_ref[0] + q_ref[1] + hs_ref[...]
    h1 = jnp.maximum(acc * dinv_ref[...], 0.0)
    out_ref[...] = jnp.dot(
        h1, w_ref[...], preferred_element_type=jnp.float32) * dinv_ref[...]

  return pl.pallas_call(
      body,
      grid=(nblk,),
      in_specs=[
          pl.BlockSpec((NC, NODE_BLK, h), lambda i: (0, i, 0)),
          pl.BlockSpec((NODE_BLK, h), lambda i: (i, 0)),
          pl.BlockSpec((NODE_BLK, 1), lambda i: (i, 0)),
          pl.BlockSpec((h, out_w), lambda i: (0, 0)),
      ],
      out_specs=pl.BlockSpec((NODE_BLK, out_w), lambda i: (i, 0)),
      out_shape=jax.ShapeDtypeStruct((n_pad, out_w), jnp.float32),
  )


def _tc_final(n_pad, h):
  nblk = n_pad // NODE_BLK

  def body(r_ref, hs_ref, dinv_ref, out_ref):
    out_ref[...] = (r_ref[0] + r_ref[1] + hs_ref[...]) * dinv_ref[...]

  return pl.pallas_call(
      body,
      grid=(nblk,),
      in_specs=[
          pl.BlockSpec((NC, NODE_BLK, h), lambda i: (0, i, 0)),
          pl.BlockSpec((NODE_BLK, h), lambda i: (i, 0)),
          pl.BlockSpec((NODE_BLK, 1), lambda i: (i, 0)),
      ],
      out_specs=pl.BlockSpec((NODE_BLK, h), lambda i: (i, 0)),
      out_shape=jax.ShapeDtypeStruct((n_pad, h), jnp.float32),
  )


def kernel(features, edge_index, W1, W2):
  n, d = features.shape
  e = edge_index.shape[1]
  h = W1.shape[1]
  out_w = W2.shape[1]
  n_pad = ((n + NODE_BLK - 1) // NODE_BLK) * NODE_BLK

  x = jnp.zeros((n_pad, d), jnp.float32).at[:n].set(features)
  nchunk = e // (NW * CHUNK)
  src2 = edge_index[0].reshape(NW, nchunk, CHUNK)
  dst2 = edge_index[1].reshape(NW, nchunk, CHUNK)
  zeros2 = jnp.zeros((n_pad, h), jnp.float32)
  zeros1 = jnp.zeros((n_pad,), jnp.float32)

  degp = _deg_pass(n_pad, e)(dst2, zeros1)
  hs1, dinv = _tc_scale_matmul(n_pad, d, h)(
      x, W1, degp.reshape(NC, n_pad, 1))
  q = _edge_pass(n_pad, h, e)(hs1, src2, dst2, zeros2)
  hs2 = _tc_mid(n_pad, h, out_w)(q, hs1, dinv, W2)
  r = _edge_pass(n_pad, out_w, e)(hs2, src2, dst2, zeros2)
  out = _tc_final(n_pad, out_w)(r, hs2, dinv)
  return out[:n]


# speedup vs baseline: 29.6502x; 1.2589x over previous
import functools

import jax
import jax.numpy as jnp
from jax import lax
from jax.experimental import pallas as pl
from jax.experimental.pallas import tpu as pltpu
from jax.experimental.pallas import tpu_sc as plsc

NC = 2
NS = 16
NW = NC * NS
CHUNK = 80
NODE_BLK = 1024


def _edge_pass(n_pad, h, e):
  epw = e // NW
  nchunk = epw // CHUNK
  rpt = n_pad // NS
  mesh = plsc.VectorSubcoreMesh(core_axis_name="c", subcore_axis_name="s")

  @functools.partial(
      pl.kernel,
      mesh=mesh,
      out_type=jax.ShapeDtypeStruct((NC, n_pad, h), jnp.float32),
      scratch_types=[
          pltpu.VMEM((nchunk, CHUNK), jnp.int32),
          pltpu.VMEM((nchunk, CHUNK), jnp.int32),
          pltpu.VMEM((CHUNK, h), jnp.float32),
          pltpu.VMEM((CHUNK, h), jnp.float32),
          pltpu.VMEM_SHARED((n_pad, h), jnp.float32),
          pltpu.SemaphoreType.DMA,
          pltpu.SemaphoreType.DMA,
      ],
      compiler_params=pltpu.CompilerParams(use_tc_tiling_on_sc=False),
  )
  def body(table_hbm, src_hbm, dst_hbm, zeros_hbm, out_hbm,
           sidx, didx, rows0, rows1, acc, sem0, sem1):
    cid = lax.axis_index("c")
    sid = lax.axis_index("s")
    wid = sid * NC + cid
    pltpu.sync_copy(src_hbm.at[wid], sidx)
    pltpu.sync_copy(dst_hbm.at[wid], didx)
    pltpu.sync_copy(zeros_hbm.at[pl.ds(sid * rpt, rpt)],
                    acc.at[pl.ds(sid * rpt, rpt)])
    plsc.subcore_barrier()

    pltpu.async_copy(table_hbm.at[sidx.at[0]], rows0, sem0).wait()
    pltpu.sync_copy(rows0, acc.at[didx.at[0]], add=True)

    def step(i, carry):
      c = 2 * i + 1
      d0 = pltpu.async_copy(table_hbm.at[sidx.at[c]], rows0, sem0)
      d1 = pltpu.async_copy(table_hbm.at[sidx.at[c + 1]], rows1, sem1)
      d0.wait()
      pltpu.sync_copy(rows0, acc.at[didx.at[c]], add=True)
      d1.wait()
      pltpu.sync_copy(rows1, acc.at[didx.at[c + 1]], add=True)
      return carry

    lax.fori_loop(0, nchunk // 2, step, 0)
    plsc.subcore_barrier()
    pltpu.sync_copy(acc.at[pl.ds(sid * rpt, rpt)],
                    out_hbm.at[cid, pl.ds(sid * rpt, rpt)])

  return body


def _deg_pass(n_pad, e):
  epw = e // NW
  nchunk = epw // CHUNK
  rpt = n_pad // NS
  mesh = plsc.VectorSubcoreMesh(core_axis_name="c", subcore_axis_name="s")

  @functools.partial(
      pl.kernel,
      mesh=mesh,
      out_type=jax.ShapeDtypeStruct((NC, n_pad), jnp.float32),
      scratch_types=[
          pltpu.VMEM((nchunk, CHUNK), jnp.int32),
          pltpu.VMEM((CHUNK,), jnp.float32),
          pltpu.VMEM_SHARED((n_pad,), jnp.float32),
      ],
      compiler_params=pltpu.CompilerParams(use_tc_tiling_on_sc=False),
  )
  def body(dst_hbm, zeros_hbm, out_hbm, didx, ones, acc):
    cid = lax.axis_index("c")
    sid = lax.axis_index("s")
    wid = sid * NC + cid
    for k in range(CHUNK // 16):
      ones[pl.ds(k * 16, 16)] = jnp.full((16,), 1.0, jnp.float32)
    pltpu.sync_copy(dst_hbm.at[wid], didx)
    pltpu.sync_copy(zeros_hbm.at[pl.ds(sid * rpt, rpt)],
                    acc.at[pl.ds(sid * rpt, rpt)])
    plsc.subcore_barrier()

    def step(c, carry):
      pltpu.sync_copy(ones, acc.at[didx.at[c]], add=True)
      return carry

    lax.fori_loop(0, nchunk, step, 0)
    plsc.subcore_barrier()
    pltpu.sync_copy(acc.at[pl.ds(sid * rpt, rpt)],
                    out_hbm.at[cid, pl.ds(sid * rpt, rpt)])

  return body


def _tc_scale_matmul(n_pad, d, h):
  nblk = n_pad // NODE_BLK

  def body(x_ref, w_ref, dp_ref, hs_ref, dinv_ref):
    deg = dp_ref[0] + dp_ref[1] + 1.0
    dinv = lax.rsqrt(jnp.maximum(deg, 1.0))
    hmat = jnp.dot(x_ref[...], w_ref[...], preferred_element_type=jnp.float32)
    hs_ref[...] = hmat * dinv
    dinv_ref[...] = dinv

  return pl.pallas_call(
      body,
      grid=(nblk,),
      in_specs=[
          pl.BlockSpec((NODE_BLK, d), lambda i: (i, 0)),
          pl.BlockSpec((d, h), lambda i: (0, 0)),
          pl.BlockSpec((NC, NODE_BLK, 1), lambda i: (0, i, 0)),
      ],
      out_specs=[
          pl.BlockSpec((NODE_BLK, h), lambda i: (i, 0)),
          pl.BlockSpec((NODE_BLK, 1), lambda i: (i, 0)),
      ],
      out_shape=[
          jax.ShapeDtypeStruct((n_pad, h), jnp.float32),
          jax.ShapeDtypeStruct((n_pad, 1), jnp.float32),
      ],
  )


def _tc_mid(n_pad, h, out_w):
  nblk = n_pad // NODE_BLK

  def body(q_ref, hs_ref, dinv_ref, w_ref, out_ref):
    acc = q_ref[0] + q_ref[1] + hs_ref[...]
    h1 = jnp.maximum(acc * dinv_ref[...], 0.0)
    out_ref[...] = jnp.dot(
        h1, w_ref[...], preferred_element_type=jnp.float32) * dinv_ref[...]

  return pl.pallas_call(
      body,
      grid=(nblk,),
      in_specs=[
          pl.BlockSpec((NC, NODE_BLK, h), lambda i: (0, i, 0)),
          pl.BlockSpec((NODE_BLK, h), lambda i: (i, 0)),
          pl.BlockSpec((NODE_BLK, 1), lambda i: (i, 0)),
          pl.BlockSpec((h, out_w), lambda i: (0, 0)),
      ],
      out_specs=pl.BlockSpec((NODE_BLK, out_w), lambda i: (i, 0)),
      out_shape=jax.ShapeDtypeStruct((n_pad, out_w), jnp.float32),
  )


def _tc_final(n_pad, h):
  nblk = n_pad // NODE_BLK

  def body(r_ref, hs_ref, dinv_ref, out_ref):
    out_ref[...] = (r_ref[0] + r_ref[1] + hs_ref[...]) * dinv_ref[...]

  return pl.pallas_call(
      body,
      grid=(nblk,),
      in_specs=[
          pl.BlockSpec((NC, NODE_BLK, h), lambda i: (0, i, 0)),
          pl.BlockSpec((NODE_BLK, h), lambda i: (i, 0)),
          pl.BlockSpec((NODE_BLK, 1), lambda i: (i, 0)),
      ],
      out_specs=pl.BlockSpec((NODE_BLK, h), lambda i: (i, 0)),
      out_shape=jax.ShapeDtypeStruct((n_pad, h), jnp.float32),
  )


def kernel(features, edge_index, W1, W2):
  n, d = features.shape
  e = edge_index.shape[1]
  h = W1.shape[1]
  out_w = W2.shape[1]
  n_pad = ((n + NODE_BLK - 1) // NODE_BLK) * NODE_BLK

  x = jnp.zeros((n_pad, d), jnp.float32).at[:n].set(features)
  nchunk = e // (NW * CHUNK)
  src2 = edge_index[0].reshape(NW, nchunk, CHUNK)
  dst2 = edge_index[1].reshape(NW, nchunk, CHUNK)
  zeros2 = jnp.zeros((n_pad, h), jnp.float32)
  zeros1 = jnp.zeros((n_pad,), jnp.float32)

  degp = _deg_pass(n_pad, e)(dst2, zeros1)
  hs1, dinv = _tc_scale_matmul(n_pad, d, h)(
      x, W1, degp.reshape(NC, n_pad, 1))
  q = _edge_pass(n_pad, h, e)(hs1, src2, dst2, zeros2)
  hs2 = _tc_mid(n_pad, h, out_w)(q, hs1, dinv, W2)
  r = _edge_pass(n_pad, out_w, e)(hs2, src2, dst2, zeros2)
  out = _tc_final(n_pad, out_w)(r, hs2, dinv)
  return out[:n]


# speedup vs baseline: 33.3841x; 1.1259x over previous
import functools

import jax
import jax.numpy as jnp
from jax import lax
from jax.experimental import pallas as pl
from jax.experimental.pallas import tpu as pltpu
from jax.experimental.pallas import tpu_sc as plsc

NC = 2
NS = 16
NW = NC * NS
CHUNK = 80
RING = 5
NODE_BLK = 1024


def _edge_pass(n_pad, h, e):
  epw = e // NW
  nchunk = epw // CHUNK
  rpt = n_pad // NS
  mesh = plsc.VectorSubcoreMesh(core_axis_name="c", subcore_axis_name="s")

  @functools.partial(
      pl.kernel,
      mesh=mesh,
      out_type=jax.ShapeDtypeStruct((NC, n_pad, h), jnp.float32),
      scratch_types=[
          pltpu.VMEM((nchunk, CHUNK), jnp.int32),
          pltpu.VMEM((nchunk, CHUNK), jnp.int32),
          [pltpu.VMEM((CHUNK, h), jnp.float32) for _ in range(RING)],
          pltpu.VMEM_SHARED((n_pad, h), jnp.float32),
          [pltpu.SemaphoreType.DMA for _ in range(RING)],
      ],
      compiler_params=pltpu.CompilerParams(use_tc_tiling_on_sc=False),
  )
  def body(table_hbm, src_hbm, dst_hbm, zeros_hbm, out_hbm,
           sidx, didx, rows, acc, sems):
    cid = lax.axis_index("c")
    sid = lax.axis_index("s")
    wid = sid * NC + cid
    pltpu.sync_copy(src_hbm.at[wid], sidx)
    pltpu.sync_copy(dst_hbm.at[wid], didx)
    pltpu.sync_copy(zeros_hbm.at[pl.ds(sid * rpt, rpt)],
                    acc.at[pl.ds(sid * rpt, rpt)])
    plsc.subcore_barrier()

    def step(i, carry):
      c = RING * i
      descs = [
          pltpu.async_copy(table_hbm.at[sidx.at[c + b]], rows[b], sems[b])
          for b in range(RING)
      ]
      for b in range(RING):
        descs[b].wait()
        pltpu.sync_copy(rows[b], acc.at[didx.at[c + b]], add=True)
      return carry

    lax.fori_loop(0, nchunk // RING, step, 0)
    plsc.subcore_barrier()
    pltpu.sync_copy(acc.at[pl.ds(sid * rpt, rpt)],
                    out_hbm.at[cid, pl.ds(sid * rpt, rpt)])

  return body


def _deg_pass(n_pad, e):
  epw = e // NW
  nchunk = epw // CHUNK
  rpt = n_pad // NS
  mesh = plsc.VectorSubcoreMesh(core_axis_name="c", subcore_axis_name="s")

  @functools.partial(
      pl.kernel,
      mesh=mesh,
      out_type=jax.ShapeDtypeStruct((NC, n_pad), jnp.float32),
      scratch_types=[
          pltpu.VMEM((nchunk, CHUNK), jnp.int32),
          pltpu.VMEM((CHUNK,), jnp.float32),
          pltpu.VMEM_SHARED((n_pad,), jnp.float32),
      ],
      compiler_params=pltpu.CompilerParams(use_tc_tiling_on_sc=False),
  )
  def body(dst_hbm, zeros_hbm, out_hbm, didx, ones, acc):
    cid = lax.axis_index("c")
    sid = lax.axis_index("s")
    wid = sid * NC + cid
    for k in range(CHUNK // 16):
      ones[pl.ds(k * 16, 16)] = jnp.full((16,), 1.0, jnp.float32)
    pltpu.sync_copy(dst_hbm.at[wid], didx)
    pltpu.sync_copy(zeros_hbm.at[pl.ds(sid * rpt, rpt)],
                    acc.at[pl.ds(sid * rpt, rpt)])
    plsc.subcore_barrier()

    def step(c, carry):
      pltpu.sync_copy(ones, acc.at[didx.at[c]], add=True)
      return carry

    lax.fori_loop(0, nchunk, step, 0)
    plsc.subcore_barrier()
    pltpu.sync_copy(acc.at[pl.ds(sid * rpt, rpt)],
                    out_hbm.at[cid, pl.ds(sid * rpt, rpt)])

  return body


def _tc_scale_matmul(n_pad, d, h):
  nblk = n_pad // NODE_BLK

  def body(x_ref, w_ref, dp_ref, hs_ref, dinv_ref):
    deg = dp_ref[0] + dp_ref[1] + 1.0
    dinv = lax.rsqrt(jnp.maximum(deg, 1.0))
    hmat = jnp.dot(x_ref[...], w_ref[...], preferred_element_type=jnp.float32)
    hs_ref[...] = hmat * dinv
    dinv_ref[...] = dinv

  return pl.pallas_call(
      body,
      grid=(nblk,),
      in_specs=[
          pl.BlockSpec((NODE_BLK, d), lambda i: (i, 0)),
          pl.BlockSpec((d, h), lambda i: (0, 0)),
          pl.BlockSpec((NC, NODE_BLK, 1), lambda i: (0, i, 0)),
      ],
      out_specs=[
          pl.BlockSpec((NODE_BLK, h), lambda i: (i, 0)),
          pl.BlockSpec((NODE_BLK, 1), lambda i: (i, 0)),
      ],
      out_shape=[
          jax.ShapeDtypeStruct((n_pad, h), jnp.float32),
          jax.ShapeDtypeStruct((n_pad, 1), jnp.float32),
      ],
  )


def _tc_mid(n_pad, h, out_w):
  nblk = n_pad // NODE_BLK

  def body(q_ref, hs_ref, dinv_ref, w_ref, out_ref):
    acc = q_ref[0] + q_ref[1] + hs_ref[...]
    h1 = jnp.maximum(acc * dinv_ref[...], 0.0)
    out_ref[...] = jnp.dot(
        h1, w_ref[...], preferred_element_type=jnp.float32) * dinv_ref[...]

  return pl.pallas_call(
      body,
      grid=(nblk,),
      in_specs=[
          pl.BlockSpec((NC, NODE_BLK, h), lambda i: (0, i, 0)),
          pl.BlockSpec((NODE_BLK, h), lambda i: (i, 0)),
          pl.BlockSpec((NODE_BLK, 1), lambda i: (i, 0)),
          pl.BlockSpec((h, out_w), lambda i: (0, 0)),
      ],
      out_specs=pl.BlockSpec((NODE_BLK, out_w), lambda i: (i, 0)),
      out_shape=jax.ShapeDtypeStruct((n_pad, out_w), jnp.float32),
  )


def _tc_final(n_pad, h):
  nblk = n_pad // NODE_BLK

  def body(r_ref, hs_ref, dinv_ref, out_ref):
    out_ref[...] = (r_ref[0] + r_ref[1] + hs_ref[...]) * dinv_ref[...]

  return pl.pallas_call(
      body,
      grid=(nblk,),
      in_specs=[
          pl.BlockSpec((NC, NODE_BLK, h), lambda i: (0, i, 0)),
          pl.BlockSpec((NODE_BLK, h), lambda i: (i, 0)),
          pl.BlockSpec((NODE_BLK, 1), lambda i: (i, 0)),
      ],
      out_specs=pl.BlockSpec((NODE_BLK, h), lambda i: (i, 0)),
      out_shape=jax.ShapeDtypeStruct((n_pad, h), jnp.float32),
  )


def kernel(features, edge_index, W1, W2):
  n, d = features.shape
  e = edge_index.shape[1]
  h = W1.shape[1]
  out_w = W2.shape[1]
  n_pad = ((n + NODE_BLK - 1) // NODE_BLK) * NODE_BLK

  x = jnp.zeros((n_pad, d), jnp.float32).at[:n].set(features)
  grain = NW * CHUNK
  e_pad = ((e + grain - 1) // grain) * grain
  pad = e_pad - e
  pad_src = jnp.arange(pad, dtype=jnp.int32) % n
  pad_dst = n + jnp.arange(pad, dtype=jnp.int32) % (n_pad - n)
  nchunk = e_pad // grain
  src2 = jnp.concatenate([edge_index[0], pad_src]).reshape(NW, nchunk, CHUNK)
  dst2 = jnp.concatenate([edge_index[1], pad_dst]).reshape(NW, nchunk, CHUNK)
  e = e_pad
  zeros2 = jnp.zeros((n_pad, h), jnp.float32)
  zeros1 = jnp.zeros((n_pad,), jnp.float32)

  degp = _deg_pass(n_pad, e)(dst2, zeros1)
  hs1, dinv = _tc_scale_matmul(n_pad, d, h)(
      x, W1, degp.reshape(NC, n_pad, 1))
  q = _edge_pass(n_pad, h, e)(hs1, src2, dst2, zeros2)
  hs2 = _tc_mid(n_pad, h, out_w)(q, hs1, dinv, W2)
  r = _edge_pass(n_pad, out_w, e)(hs2, src2, dst2, zeros2)
  out = _tc_final(n_pad, out_w)(r, hs2, dinv)
  return out[:n]


# speedup vs baseline: 33.4108x; 1.0008x over previous
import functools

import jax
import jax.numpy as jnp
from jax import lax
from jax.experimental import pallas as pl
from jax.experimental.pallas import tpu as pltpu
from jax.experimental.pallas import tpu_sc as plsc

NC = 2
NS = 16
NW = NC * NS
CHUNK = 80
RING = 5
NODE_BLK = 1024


def _edge_pass(n_pad, h, e):
  epw = e // NW
  nchunk = epw // CHUNK
  rpt = n_pad // NS
  mesh = plsc.VectorSubcoreMesh(core_axis_name="c", subcore_axis_name="s")

  @functools.partial(
      pl.kernel,
      mesh=mesh,
      out_type=jax.ShapeDtypeStruct((NC, n_pad, h), jnp.float32),
      scratch_types=[
          pltpu.VMEM((nchunk, CHUNK), jnp.int32),
          pltpu.VMEM((nchunk, CHUNK), jnp.int32),
          [pltpu.VMEM((CHUNK, h), jnp.float32) for _ in range(RING)],
          pltpu.VMEM_SHARED((n_pad, h), jnp.float32),
          [pltpu.SemaphoreType.DMA for _ in range(RING)],
      ],
      compiler_params=pltpu.CompilerParams(use_tc_tiling_on_sc=False),
  )
  def body(table_hbm, src_hbm, dst_hbm, zeros_hbm, out_hbm,
           sidx, didx, rows, acc, sems):
    cid = lax.axis_index("c")
    sid = lax.axis_index("s")
    wid = sid * NC + cid
    pltpu.sync_copy(src_hbm.at[wid], sidx)
    pltpu.sync_copy(dst_hbm.at[wid], didx)
    pltpu.sync_copy(zeros_hbm.at[pl.ds(sid * rpt, rpt)],
                    acc.at[pl.ds(sid * rpt, rpt)])
    plsc.subcore_barrier()

    def step(i, carry):
      c = RING * i
      descs = [
          pltpu.async_copy(table_hbm.at[sidx.at[c + b]], rows[b], sems[b])
          for b in range(RING)
      ]
      for b in range(RING):
        descs[b].wait()
        pltpu.sync_copy(rows[b], acc.at[didx.at[c + b]], add=True)
      return carry

    lax.fori_loop(0, nchunk // RING, step, 0)
    plsc.subcore_barrier()
    pltpu.sync_copy(acc.at[pl.ds(sid * rpt, rpt)],
                    out_hbm.at[cid, pl.ds(sid * rpt, rpt)])

  return body


def _deg_pass(n_pad, e):
  epw = e // NW
  nchunk = epw // CHUNK
  rpt = n_pad // NS
  mesh = plsc.VectorSubcoreMesh(core_axis_name="c", subcore_axis_name="s")

  @functools.partial(
      pl.kernel,
      mesh=mesh,
      out_type=jax.ShapeDtypeStruct((NC, n_pad), jnp.float32),
      scratch_types=[
          pltpu.VMEM((nchunk, CHUNK), jnp.int32),
          pltpu.VMEM((CHUNK,), jnp.float32),
          pltpu.VMEM_SHARED((n_pad,), jnp.float32),
      ],
      compiler_params=pltpu.CompilerParams(use_tc_tiling_on_sc=False),
  )
  def body(dst_hbm, zeros_hbm, out_hbm, didx, ones, acc):
    cid = lax.axis_index("c")
    sid = lax.axis_index("s")
    wid = sid * NC + cid
    for k in range(CHUNK // 16):
      ones[pl.ds(k * 16, 16)] = jnp.full((16,), 1.0, jnp.float32)
    pltpu.sync_copy(dst_hbm.at[wid], didx)
    pltpu.sync_copy(zeros_hbm.at[pl.ds(sid * rpt, rpt)],
                    acc.at[pl.ds(sid * rpt, rpt)])
    plsc.subcore_barrier()

    def step(c, carry):
      pltpu.sync_copy(ones, acc.at[didx.at[c]], add=True)
      return carry

    lax.fori_loop(0, nchunk, step, 0)
    plsc.subcore_barrier()
    pltpu.sync_copy(acc.at[pl.ds(sid * rpt, rpt)],
                    out_hbm.at[cid, pl.ds(sid * rpt, rpt)])

  return body


def _tc_matmul(n_pad, d, h):
  nblk = n_pad // NODE_BLK

  def body(x_ref, w_ref, h_ref):
    h_ref[...] = jnp.dot(
        x_ref[...], w_ref[...], preferred_element_type=jnp.float32)

  return pl.pallas_call(
      body,
      grid=(nblk,),
      in_specs=[
          pl.BlockSpec((NODE_BLK, d), lambda i: (i, 0)),
          pl.BlockSpec((d, h), lambda i: (0, 0)),
      ],
      out_specs=pl.BlockSpec((NODE_BLK, h), lambda i: (i, 0)),
      out_shape=jax.ShapeDtypeStruct((n_pad, h), jnp.float32),
  )


def _tc_scale(n_pad, h):
  nblk = n_pad // NODE_BLK

  def body(h_ref, dp_ref, hs_ref, dinv_ref):
    deg = dp_ref[0] + dp_ref[1] + 1.0
    dinv = lax.rsqrt(jnp.maximum(deg, 1.0))
    hs_ref[...] = h_ref[...] * dinv
    dinv_ref[...] = dinv

  return pl.pallas_call(
      body,
      grid=(nblk,),
      in_specs=[
          pl.BlockSpec((NODE_BLK, h), lambda i: (i, 0)),
          pl.BlockSpec((NC, NODE_BLK, 1), lambda i: (0, i, 0)),
      ],
      out_specs=[
          pl.BlockSpec((NODE_BLK, h), lambda i: (i, 0)),
          pl.BlockSpec((NODE_BLK, 1), lambda i: (i, 0)),
      ],
      out_shape=[
          jax.ShapeDtypeStruct((n_pad, h), jnp.float32),
          jax.ShapeDtypeStruct((n_pad, 1), jnp.float32),
      ],
  )


def _tc_mid(n_pad, h, out_w):
  nblk = n_pad // NODE_BLK

  def body(q_ref, hs_ref, dinv_ref, w_ref, out_ref):
    acc = q_ref[0] + q_ref[1] + hs_ref[...]
    h1 = jnp.maximum(acc * dinv_ref[...], 0.0)
    out_ref[...] = jnp.dot(
        h1, w_ref[...], preferred_element_type=jnp.float32) * dinv_ref[...]

  return pl.pallas_call(
      body,
      grid=(nblk,),
      in_specs=[
          pl.BlockSpec((NC, NODE_BLK, h), lambda i: (0, i, 0)),
          pl.BlockSpec((NODE_BLK, h), lambda i: (i, 0)),
          pl.BlockSpec((NODE_BLK, 1), lambda i: (i, 0)),
          pl.BlockSpec((h, out_w), lambda i: (0, 0)),
      ],
      out_specs=pl.BlockSpec((NODE_BLK, out_w), lambda i: (i, 0)),
      out_shape=jax.ShapeDtypeStruct((n_pad, out_w), jnp.float32),
  )


def _tc_final(n_pad, h):
  nblk = n_pad // NODE_BLK

  def body(r_ref, hs_ref, dinv_ref, out_ref):
    out_ref[...] = (r_ref[0] + r_ref[1] + hs_ref[...]) * dinv_ref[...]

  return pl.pallas_call(
      body,
      grid=(nblk,),
      in_specs=[
          pl.BlockSpec((NC, NODE_BLK, h), lambda i: (0, i, 0)),
          pl.BlockSpec((NODE_BLK, h), lambda i: (i, 0)),
          pl.BlockSpec((NODE_BLK, 1), lambda i: (i, 0)),
      ],
      out_specs=pl.BlockSpec((NODE_BLK, h), lambda i: (i, 0)),
      out_shape=jax.ShapeDtypeStruct((n_pad, h), jnp.float32),
  )


def kernel(features, edge_index, W1, W2):
  n, d = features.shape
  e = edge_index.shape[1]
  h = W1.shape[1]
  out_w = W2.shape[1]
  n_pad = ((n + NODE_BLK - 1) // NODE_BLK) * NODE_BLK

  x = jnp.zeros((n_pad, d), jnp.float32).at[:n].set(features)
  grain = NW * CHUNK
  e_pad = ((e + grain - 1) // grain) * grain
  pad = e_pad - e
  pad_src = jnp.arange(pad, dtype=jnp.int32) % n
  pad_dst = n + jnp.arange(pad, dtype=jnp.int32) % (n_pad - n)
  nchunk = e_pad // grain
  src2 = jnp.concatenate([edge_index[0], pad_src]).reshape(NW, nchunk, CHUNK)
  dst2 = jnp.concatenate([edge_index[1], pad_dst]).reshape(NW, nchunk, CHUNK)
  e = e_pad
  zeros2 = jnp.zeros((n_pad, h), jnp.float32)
  zeros1 = jnp.zeros((n_pad,), jnp.float32)

  degp = _deg_pass(n_pad, e)(dst2, zeros1)
  hmat = _tc_matmul(n_pad, d, h)(x, W1)
  hs1, dinv = _tc_scale(n_pad, h)(hmat, degp.reshape(NC, n_pad, 1))
  q = _edge_pass(n_pad, h, e)(hs1, src2, dst2, zeros2)
  hs2 = _tc_mid(n_pad, h, out_w)(q, hs1, dinv, W2)
  r = _edge_pass(n_pad, out_w, e)(hs2, src2, dst2, zeros2)
  out = _tc_final(n_pad, out_w)(r, hs2, dinv)
  return out[:n]


# speedup vs baseline: 34.7334x; 1.0396x over previous
import functools

import jax
import jax.numpy as jnp
from jax import lax
from jax.experimental import pallas as pl
from jax.experimental.pallas import tpu as pltpu
from jax.experimental.pallas import tpu_sc as plsc

NC = 2
NS = 16
NW = NC * NS
CHUNK = 80
RING = 5
NODE_BLK = 1024


def _edge_pass(n_pad, h, e):
  epw = e // NW
  nchunk = epw // CHUNK
  rpt = n_pad // NS
  mesh = plsc.VectorSubcoreMesh(core_axis_name="c", subcore_axis_name="s")

  @functools.partial(
      pl.kernel,
      mesh=mesh,
      out_type=jax.ShapeDtypeStruct((NC, n_pad, h), jnp.float32),
      scratch_types=[
          pltpu.VMEM((nchunk, CHUNK), jnp.int32),
          pltpu.VMEM((nchunk, CHUNK), jnp.int32),
          [pltpu.VMEM((CHUNK, h), jnp.float32) for _ in range(RING)],
          pltpu.VMEM_SHARED((n_pad, h), jnp.float32),
          [pltpu.SemaphoreType.DMA for _ in range(RING)],
      ],
      compiler_params=pltpu.CompilerParams(use_tc_tiling_on_sc=False),
  )
  def body(table_hbm, src_hbm, dst_hbm, zeros_hbm, out_hbm,
           sidx, didx, rows, acc, sems):
    cid = lax.axis_index("c")
    sid = lax.axis_index("s")
    wid = sid * NC + cid
    pltpu.sync_copy(src_hbm.at[wid], sidx)
    pltpu.sync_copy(dst_hbm.at[wid], didx)
    pltpu.sync_copy(zeros_hbm.at[pl.ds(sid * rpt, rpt)],
                    acc.at[pl.ds(sid * rpt, rpt)])
    plsc.subcore_barrier()

    def step(i, carry):
      c = RING * i
      descs = [
          pltpu.async_copy(table_hbm.at[sidx.at[c + b]], rows[b], sems[b])
          for b in range(RING)
      ]
      for b in range(RING):
        descs[b].wait()
        pltpu.sync_copy(rows[b], acc.at[didx.at[c + b]], add=True)
      return carry

    lax.fori_loop(0, nchunk // RING, step, 0)
    plsc.subcore_barrier()
    pltpu.sync_copy(acc.at[pl.ds(sid * rpt, rpt)],
                    out_hbm.at[cid, pl.ds(sid * rpt, rpt)])

  return body


def _deg_pass(n_pad, e):
  epw = e // NW
  nchunk = epw // CHUNK
  rpt = n_pad // NS
  mesh = plsc.VectorSubcoreMesh(core_axis_name="c", subcore_axis_name="s")

  @functools.partial(
      pl.kernel,
      mesh=mesh,
      out_type=jax.ShapeDtypeStruct((NC, n_pad), jnp.float32),
      scratch_types=[
          pltpu.VMEM((nchunk, CHUNK), jnp.int32),
          pltpu.VMEM((CHUNK,), jnp.float32),
          pltpu.VMEM_SHARED((n_pad,), jnp.float32),
      ],
      compiler_params=pltpu.CompilerParams(use_tc_tiling_on_sc=False),
  )
  def body(dst_hbm, zeros_hbm, out_hbm, didx, ones, acc):
    cid = lax.axis_index("c")
    sid = lax.axis_index("s")
    wid = sid * NC + cid
    for k in range(CHUNK // 16):
      ones[pl.ds(k * 16, 16)] = jnp.full((16,), 1.0, jnp.float32)
    pltpu.sync_copy(dst_hbm.at[wid], didx)
    pltpu.sync_copy(zeros_hbm.at[pl.ds(sid * rpt, rpt)],
                    acc.at[pl.ds(sid * rpt, rpt)])
    plsc.subcore_barrier()

    def step(c, carry):
      pltpu.sync_copy(ones, acc.at[didx.at[c]], add=True)
      return carry

    lax.fori_loop(0, nchunk, step, 0)
    plsc.subcore_barrier()
    pltpu.sync_copy(acc.at[pl.ds(sid * rpt, rpt)],
                    out_hbm.at[cid, pl.ds(sid * rpt, rpt)])

  return body


def _tc_matmul(n_pad, d, h):
  nblk = n_pad // NODE_BLK

  def body(x_ref, w_ref, h_ref):
    h_ref[...] = jnp.dot(
        x_ref[...], w_ref[...], preferred_element_type=jnp.float32)

  return pl.pallas_call(
      body,
      grid=(nblk,),
      in_specs=[
          pl.BlockSpec((NODE_BLK, d), lambda i: (i, 0)),
          pl.BlockSpec((d, h), lambda i: (0, 0)),
      ],
      out_specs=pl.BlockSpec((NODE_BLK, h), lambda i: (i, 0)),
      out_shape=jax.ShapeDtypeStruct((n_pad, h), jnp.float32),
  )


def _tc_scale(n_pad, h):
  nblk = n_pad // NODE_BLK

  def body(h_ref, dp_ref, hs_ref, dinv_ref):
    deg = dp_ref[0] + dp_ref[1] + 1.0
    dinv = lax.rsqrt(jnp.maximum(deg, 1.0))
    dcol = jnp.transpose(dinv.reshape(1, -1), (1, 0))
    hs_ref[...] = h_ref[...] * dcol
    dinv_ref[...] = dcol

  return pl.pallas_call(
      body,
      grid=(nblk,),
      in_specs=[
          pl.BlockSpec((NODE_BLK, h), lambda i: (i, 0)),
          pl.BlockSpec((NC, NODE_BLK), lambda i: (0, i)),
      ],
      out_specs=[
          pl.BlockSpec((NODE_BLK, h), lambda i: (i, 0)),
          pl.BlockSpec((NODE_BLK, 1), lambda i: (i, 0)),
      ],
      out_shape=[
          jax.ShapeDtypeStruct((n_pad, h), jnp.float32),
          jax.ShapeDtypeStruct((n_pad, 1), jnp.float32),
      ],
  )


def _tc_mid(n_pad, h, out_w):
  nblk = n_pad // NODE_BLK

  def body(q_ref, hs_ref, dinv_ref, w_ref, out_ref):
    acc = q_ref[0] + q_ref[1] + hs_ref[...]
    h1 = jnp.maximum(acc * dinv_ref[...], 0.0)
    out_ref[...] = jnp.dot(
        h1, w_ref[...], preferred_element_type=jnp.float32) * dinv_ref[...]

  return pl.pallas_call(
      body,
      grid=(nblk,),
      in_specs=[
          pl.BlockSpec((NC, NODE_BLK, h), lambda i: (0, i, 0)),
          pl.BlockSpec((NODE_BLK, h), lambda i: (i, 0)),
          pl.BlockSpec((NODE_BLK, 1), lambda i: (i, 0)),
          pl.BlockSpec((h, out_w), lambda i: (0, 0)),
      ],
      out_specs=pl.BlockSpec((NODE_BLK, out_w), lambda i: (i, 0)),
      out_shape=jax.ShapeDtypeStruct((n_pad, out_w), jnp.float32),
  )


def _tc_final(n_pad, h):
  nblk = n_pad // NODE_BLK

  def body(r_ref, hs_ref, dinv_ref, out_ref):
    out_ref[...] = (r_ref[0] + r_ref[1] + hs_ref[...]) * dinv_ref[...]

  return pl.pallas_call(
      body,
      grid=(nblk,),
      in_specs=[
          pl.BlockSpec((NC, NODE_BLK, h), lambda i: (0, i, 0)),
          pl.BlockSpec((NODE_BLK, h), lambda i: (i, 0)),
          pl.BlockSpec((NODE_BLK, 1), lambda i: (i, 0)),
      ],
      out_specs=pl.BlockSpec((NODE_BLK, h), lambda i: (i, 0)),
      out_shape=jax.ShapeDtypeStruct((n_pad, h), jnp.float32),
  )


def kernel(features, edge_index, W1, W2):
  n, d = features.shape
  e = edge_index.shape[1]
  h = W1.shape[1]
  out_w = W2.shape[1]
  n_pad = ((n + NODE_BLK - 1) // NODE_BLK) * NODE_BLK

  x = jnp.zeros((n_pad, d), jnp.float32).at[:n].set(features)
  grain = NW * CHUNK
  e_pad = ((e + grain - 1) // grain) * grain
  pad = e_pad - e
  pad_src = jnp.arange(pad, dtype=jnp.int32) % n
  pad_dst = n + jnp.arange(pad, dtype=jnp.int32) % (n_pad - n)
  nchunk = e_pad // grain
  src2 = jnp.concatenate([edge_index[0], pad_src]).reshape(NW, nchunk, CHUNK)
  dst2 = jnp.concatenate([edge_index[1], pad_dst]).reshape(NW, nchunk, CHUNK)
  e = e_pad
  zeros2 = jnp.zeros((n_pad, h), jnp.float32)
  zeros1 = jnp.zeros((n_pad,), jnp.float32)

  degp = _deg_pass(n_pad, e)(dst2, zeros1)
  hmat = _tc_matmul(n_pad, d, h)(x, W1)
  hs1, dinv = _tc_scale(n_pad, h)(hmat, degp)
  q = _edge_pass(n_pad, h, e)(hs1, src2, dst2, zeros2)
  hs2 = _tc_mid(n_pad, h, out_w)(q, hs1, dinv, W2)
  r = _edge_pass(n_pad, out_w, e)(hs2, src2, dst2, zeros2)
  out = _tc_final(n_pad, out_w)(r, hs2, dinv)
  return out[:n]
